# baseline (speedup 1.0000x reference)
import functools

import jax
import jax.numpy as jnp
from jax import lax
from jax.experimental import pallas as pl
from jax.experimental.pallas import tpu as pltpu
from jax.experimental.pallas import tpu_sc as plsc

B = 8
N = 4096
S = 512
K = 32
D = 64
MLP_CH = [64, 64, 128]
EPS = 1e-5


def _fps_body(x_ref, y_ref, z_ref, nx_ref, ny_ref, nz_ref, dist_ref):
    x = x_ref[...]
    y = y_ref[...]
    z = z_ref[...]
    iota = jax.lax.broadcasted_iota(jnp.int32, (B, N), 1)
    lane = jax.lax.broadcasted_iota(jnp.int32, (B, 128), 1)
    dist_ref[...] = jnp.full((B, N), 1e10, jnp.float32)

    def body(i, state):
        far, bx, by, bz = state
        onehot = iota == far
        cx = jnp.max(jnp.where(onehot, x, -jnp.inf), axis=1, keepdims=True)
        cy = jnp.max(jnp.where(onehot, y, -jnp.inf), axis=1, keepdims=True)
        cz = jnp.max(jnp.where(onehot, z, -jnp.inf), axis=1, keepdims=True)
        sel = lane == i
        bx = jnp.where(sel, cx, bx)
        by = jnp.where(sel, cy, by)
        bz = jnp.where(sel, cz, bz)
        dx = x - cx
        dy = y - cy
        dz = z - cz
        d = dx * dx + dy * dy + dz * dz
        dmin = jnp.minimum(dist_ref[...], d)
        dist_ref[...] = dmin
        m = jnp.max(dmin, axis=1, keepdims=True)
        far_new = jnp.min(jnp.where(dmin == m, iota, N), axis=1, keepdims=True)
        return far_new, bx, by, bz

    far = jnp.zeros((B, 1), jnp.int32)
    zbuf = jnp.zeros((B, 128), jnp.float32)
    for j in range(S // 128):
        far, bx, by, bz = jax.lax.fori_loop(0, 128, body, (far, zbuf, zbuf, zbuf))
        nx_ref[:, j * 128:(j + 1) * 128] = bx
        ny_ref[:, j * 128:(j + 1) * 128] = by
        nz_ref[:, j * 128:(j + 1) * 128] = bz


def _fps(x, y, z):
    out = pl.pallas_call(
        _fps_body,
        out_shape=[jax.ShapeDtypeStruct((B, S), jnp.float32)] * 3,
        scratch_shapes=[pltpu.VMEM((B, N), jnp.float32)],
    )(x, y, z)
    return out


SBLK = 128


def _knn_body(x_ref, y_ref, z_ref, cx_ref, cy_ref, cz_ref, di_ref, thr_ref):
    x = x_ref[0]
    cx = cx_ref[0]
    dx = x - cx
    dy = y_ref[0] - cy_ref[0]
    dz = z_ref[0] - cz_ref[0]
    d = dx * dx + dy * dy + dz * dz
    di = jax.lax.bitcast_convert_type(d, jnp.int32)
    di_ref[...] = di[None]
    acc = jnp.zeros((SBLK, 1), jnp.int32)
    for b in range(30, -1, -1):
        trial = acc | (1 << b)
        cnt = jnp.sum((di < trial).astype(jnp.int32), axis=1, keepdims=True)
        acc = jnp.where(cnt < K, trial, acc)
    thr_ref[...] = acc[None]


def _knn_thresh(x, y, z, cxg, cyg, czg):
    nsb = S // SBLK
    grid = (B, nsb)
    return pl.pallas_call(
        _knn_body,
        grid=grid,
        in_specs=[
            pl.BlockSpec((1, 1, N), lambda b, s: (b, 0, 0)),
            pl.BlockSpec((1, 1, N), lambda b, s: (b, 0, 0)),
            pl.BlockSpec((1, 1, N), lambda b, s: (b, 0, 0)),
            pl.BlockSpec((1, SBLK, 1), lambda b, s: (b * nsb + s, 0, 0)),
            pl.BlockSpec((1, SBLK, 1), lambda b, s: (b * nsb + s, 0, 0)),
            pl.BlockSpec((1, SBLK, 1), lambda b, s: (b * nsb + s, 0, 0)),
        ],
        out_specs=[
            pl.BlockSpec((1, SBLK, N), lambda b, s: (b, s, 0)),
            pl.BlockSpec((1, SBLK, 1), lambda b, s: (b * nsb + s, 0, 0)),
        ],
        out_shape=[
            jax.ShapeDtypeStruct((B, S, N), jnp.int32),
            jax.ShapeDtypeStruct((B * nsb, SBLK, 1), jnp.int32),
        ],
    )(x, y, z, cxg, cyg, czg)


XP = 16


def _sc_group_gather(di_f, thr, pts, xyzp, cen):
    info = plsc.get_sparse_core_info()
    NC, NS = info.num_cores, info.num_subcores
    NW = NC * NS
    rpw = (B * S) // NW
    mesh = plsc.VectorSubcoreMesh(core_axis_name="c", subcore_axis_name="s")

    @functools.partial(
        pl.kernel, mesh=mesh,
        compiler_params=pltpu.CompilerParams(
            needs_layout_passes=False, use_tc_tiling_on_sc=False),
        out_type=[jax.ShapeDtypeStruct((B * S, K, D), jnp.float32),
                  jax.ShapeDtypeStruct((B * S, K, XP), jnp.float32)],
        scratch_types=[
            pltpu.VMEM((N,), jnp.int32),
            pltpu.VMEM((rpw,), jnp.int32),
            pltpu.VMEM((K,), jnp.int32),
            pltpu.VMEM((K,), jnp.int32),
            pltpu.VMEM((K, D), jnp.float32),
            pltpu.VMEM((K, XP), jnp.float32),
            pltpu.VMEM((XP,), jnp.float32),
            pltpu.SemaphoreType.DMA,
        ],
    )
    def k(di_hbm, thr_hbm, pts_hbm, xyzp_hbm, cen_hbm, gp_hbm, gx_hbm,
          dirow, thrv_ref, selg, eqbuf, prow, xrow, cenv, sem):
        wid = lax.axis_index("s") * NC + lax.axis_index("c")
        base = wid * rpw
        pltpu.sync_copy(thr_hbm.at[pl.ds(base, rpw)], thrv_ref)
        iota16 = lax.iota(jnp.int32, 16)

        def row_body(r, carry):
            g = base + r
            b = g // S
            bn = b * N
            pltpu.sync_copy(di_hbm.at[g], dirow)
            thrv = plsc.load_gather(thrv_ref, [jnp.full((16,), r, jnp.int32)])

            def chunk(c, offs):
                v = dirow[pl.ds(c * 16, 16)]
                mlt = v < thrv
                meq = v == thrv

                def do(offs):
                    offlt, offeq = offs
                    gidx = c * 16 + iota16 + bn
                    rlt = offlt + plsc.cumsum(mlt.astype(jnp.int32)) - 1
                    plsc.store_scatter(selg, [rlt], gidx, mask=mlt)
                    req = offeq + plsc.cumsum(meq.astype(jnp.int32)) - 1
                    meq2 = meq & (req < K)
                    plsc.store_scatter(eqbuf, [req], gidx, mask=meq2)
                    return (offlt + plsc.all_reduce_population_count(mlt),
                            offeq + plsc.all_reduce_population_count(meq2))

                return lax.cond(jnp.any(mlt | meq), do, lambda o: o, offs)

            z16 = jnp.zeros((16,), jnp.int32)
            nlt, _ = lax.fori_loop(0, N // 16, chunk, (z16, z16))
            for t in range(K // 16):
                e = eqbuf[pl.ds(t * 16, 16)]
                i = t * 16 + iota16
                plsc.store_scatter(selg, [nlt + i], e, mask=i < (K - nlt))

            pltpu.async_copy(pts_hbm.at[selg], prow, sem).wait()
            pltpu.sync_copy(prow, gp_hbm.at[g])
            pltpu.async_copy(xyzp_hbm.at[selg], xrow, sem).wait()
            pltpu.sync_copy(cen_hbm.at[g], cenv)
            cv = cenv[...]
            for j in range(K):
                xrow[j] = xrow[j] - cv
            pltpu.sync_copy(xrow, gx_hbm.at[g])
            return carry

        lax.fori_loop(0, rpw, row_body, 0)

    return k(di_f, thr, pts, xyzp, cen)


M = B * S * K
MB = 4096


def _stats_pad(y, oc):
    s = jnp.sum(y, axis=0, keepdims=True)
    q = jnp.sum(y * y, axis=0, keepdims=True)
    if oc < 128:
        z = jnp.zeros((1, 128 - oc), jnp.float32)
        s = jnp.concatenate([s, z], axis=1)
        q = jnp.concatenate([q, z], axis=1)
    return jnp.concatenate([s, q, jnp.zeros((6, 128), jnp.float32)], axis=0)


def _acc_stats(st_ref, st):
    @pl.when(pl.program_id(0) == 0)
    def _():
        st_ref[...] = jnp.zeros((8, 128), jnp.float32)

    st_ref[...] += st


def _l1_body(gp_ref, gx_ref, wp_ref, wx_ref, y_ref, st_ref):
    y = jnp.dot(gp_ref[...], wp_ref[...], preferred_element_type=jnp.float32)
    y = y + jnp.dot(gx_ref[...], wx_ref[...],
                    preferred_element_type=jnp.float32)
    y_ref[...] = y
    _acc_stats(st_ref, _stats_pad(y, 64))


def _layer1(gp, gx, wp, wx):
    return pl.pallas_call(
        _l1_body,
        grid=(M // MB,),
        in_specs=[
            pl.BlockSpec((MB, D), lambda i: (i, 0)),
            pl.BlockSpec((MB, XP), lambda i: (i, 0)),
            pl.BlockSpec((D, 64), lambda i: (0, 0)),
            pl.BlockSpec((XP, 64), lambda i: (0, 0)),
        ],
        out_specs=[
            pl.BlockSpec((MB, 64), lambda i: (i, 0)),
            pl.BlockSpec((8, 128), lambda i: (0, 0)),
        ],
        out_shape=[
            jax.ShapeDtypeStruct((M, 64), jnp.float32),
            jax.ShapeDtypeStruct((8, 128), jnp.float32),
        ],
    )(gp, gx, wp, wx)


def _bn_affine(st, g, be, ic):
    mean = st[0:1, :ic] * (1.0 / M)
    var = st[1:2, :ic] * (1.0 / M) - mean * mean
    a = g / jnp.sqrt(var + EPS)
    c = be - mean * a
    return a, c


def _mid_body(ic, oc, y_ref, st_ref, g_ref, be_ref, w_ref, o_ref, so_ref):
    a, c = _bn_affine(st_ref[...], g_ref[...], be_ref[...], ic)
    z = jnp.maximum(y_ref[...] * a + c, 0.0)
    o = jnp.dot(z, w_ref[...], preferred_element_type=jnp.float32)
    o_ref[...] = o
    _acc_stats(so_ref, _stats_pad(o, oc))


def _mid_layer(y, st, g, be, w, ic, oc):
    return pl.pallas_call(
        functools.partial(_mid_body, ic, oc),
        grid=(M // MB,),
        in_specs=[
            pl.BlockSpec((MB, ic), lambda i: (i, 0)),
            pl.BlockSpec((8, 128), lambda i: (0, 0)),
            pl.BlockSpec((1, ic), lambda i: (0, 0)),
            pl.BlockSpec((1, ic), lambda i: (0, 0)),
            pl.BlockSpec((ic, oc), lambda i: (0, 0)),
        ],
        out_specs=[
            pl.BlockSpec((MB, oc), lambda i: (i, 0)),
            pl.BlockSpec((8, 128), lambda i: (0, 0)),
        ],
        out_shape=[
            jax.ShapeDtypeStruct((M, oc), jnp.float32),
            jax.ShapeDtypeStruct((8, 128), jnp.float32),
        ],
    )(y, st, g, be, w)


def _pool_body(y_ref, st_ref, g_ref, be_ref, o_ref):
    a, c = _bn_affine(st_ref[...], g_ref[...], be_ref[...], 128)
    z = jnp.maximum(y_ref[...] * a + c, 0.0)
    o_ref[...] = jnp.max(z.reshape(MB // K, K, 128), axis=1)


def _pool_layer(y, st, g, be):
    return pl.pallas_call(
        _pool_body,
        grid=(M // MB,),
        in_specs=[
            pl.BlockSpec((MB, 128), lambda i: (i, 0)),
            pl.BlockSpec((8, 128), lambda i: (0, 0)),
            pl.BlockSpec((1, 128), lambda i: (0, 0)),
            pl.BlockSpec((1, 128), lambda i: (0, 0)),
        ],
        out_specs=pl.BlockSpec((MB // K, 128), lambda i: (i, 0)),
        out_shape=jax.ShapeDtypeStruct((M // K, 128), jnp.float32),
    )(y, st, g, be)


def kernel(xyz, points, params):
    x = xyz[:, :, 0]
    y = xyz[:, :, 1]
    z = xyz[:, :, 2]
    nx, ny, nz = _fps(x, y, z)
    new_xyz = jnp.stack([nx, ny, nz], axis=-1)

    def _cg(a):
        return a.reshape(B * (S // SBLK), SBLK, 1)

    di, thrg = _knn_thresh(x[:, None, :], y[:, None, :], z[:, None, :],
                           _cg(nx), _cg(ny), _cg(nz))
    thr = thrg.reshape(B, S)

    xyzp = jnp.pad(xyz, ((0, 0), (0, 0), (0, XP - 3))).reshape(B * N, XP)
    cen = jnp.pad(new_xyz, ((0, 0), (0, 0), (0, XP - 3))).reshape(B * S, XP)
    gp, gx = _sc_group_gather(
        di.reshape(B * S, N), thr.reshape(B * S), points.reshape(B * N, D),
        xyzp, cen)
    w0 = params["w0"]
    wx = jnp.pad(w0[:, :3].T, ((0, XP - 3), (0, 0)))
    wp = w0[:, 3:].T
    y1, st1 = _layer1(gp.reshape(M, D), gx.reshape(M, XP), wp, wx)
    y2, st2 = _mid_layer(y1, st1, params["g0"][None, :], params["be0"][None, :],
                         params["w1"].T, 64, 64)
    y3, st3 = _mid_layer(y2, st2, params["g1"][None, :], params["be1"][None, :],
                         params["w2"].T, 64, 128)
    out = _pool_layer(y3, st3, params["g2"][None, :], params["be2"][None, :])
    return (new_xyz, out.reshape(B, S, 128))


# speedup vs baseline: 1.2885x; 1.2885x over previous
import functools

import jax
import jax.numpy as jnp
from jax import lax
from jax.experimental import pallas as pl
from jax.experimental.pallas import tpu as pltpu
from jax.experimental.pallas import tpu_sc as plsc

B = 8
N = 4096
S = 512
K = 32
D = 64
MLP_CH = [64, 64, 128]
EPS = 1e-5


def _fps_body(x_ref, y_ref, z_ref, nx_ref, ny_ref, nz_ref, dist_ref):
    x = x_ref[...]
    y = y_ref[...]
    z = z_ref[...]
    iota = jax.lax.broadcasted_iota(jnp.int32, (B, N), 1)
    lane = jax.lax.broadcasted_iota(jnp.int32, (B, 128), 1)
    dist_ref[...] = jnp.full((B, N), 1e10, jnp.float32)

    def body(i, state):
        far, bx, by, bz = state
        onehot = iota == far
        cx = jnp.max(jnp.where(onehot, x, -jnp.inf), axis=1, keepdims=True)
        cy = jnp.max(jnp.where(onehot, y, -jnp.inf), axis=1, keepdims=True)
        cz = jnp.max(jnp.where(onehot, z, -jnp.inf), axis=1, keepdims=True)
        sel = lane == i
        bx = jnp.where(sel, cx, bx)
        by = jnp.where(sel, cy, by)
        bz = jnp.where(sel, cz, bz)
        dx = x - cx
        dy = y - cy
        dz = z - cz
        d = dx * dx + dy * dy + dz * dz
        dmin = jnp.minimum(dist_ref[...], d)
        dist_ref[...] = dmin
        m = jnp.max(dmin, axis=1, keepdims=True)
        far_new = jnp.min(jnp.where(dmin == m, iota, N), axis=1, keepdims=True)
        return far_new, bx, by, bz

    far = jnp.zeros((B, 1), jnp.int32)
    zbuf = jnp.zeros((B, 128), jnp.float32)
    for j in range(S // 128):
        far, bx, by, bz = jax.lax.fori_loop(0, 128, body, (far, zbuf, zbuf, zbuf))
        nx_ref[:, j * 128:(j + 1) * 128] = bx
        ny_ref[:, j * 128:(j + 1) * 128] = by
        nz_ref[:, j * 128:(j + 1) * 128] = bz


def _fps(x, y, z):
    out = pl.pallas_call(
        _fps_body,
        out_shape=[jax.ShapeDtypeStruct((B, S), jnp.float32)] * 3,
        scratch_shapes=[pltpu.VMEM((B, N), jnp.float32)],
    )(x, y, z)
    return out


SBLK = 128


def _knn_body(x_ref, y_ref, z_ref, cx_ref, cy_ref, cz_ref, di_ref, thr_ref):
    x = x_ref[0]
    cx = cx_ref[0]
    dx = x - cx
    dy = y_ref[0] - cy_ref[0]
    dz = z_ref[0] - cz_ref[0]
    d = dx * dx + dy * dy + dz * dz
    di = jax.lax.bitcast_convert_type(d, jnp.int32)
    di_ref[...] = di[None]
    acc = jnp.zeros((SBLK, 1), jnp.int32)
    for b in range(30, -1, -1):
        trial = acc | (1 << b)
        cnt = jnp.sum((di < trial).astype(jnp.int32), axis=1, keepdims=True)
        acc = jnp.where(cnt < K, trial, acc)
    thr_ref[...] = acc[None]


def _knn_thresh(x, y, z, cxg, cyg, czg):
    nsb = S // SBLK
    grid = (B, nsb)
    return pl.pallas_call(
        _knn_body,
        grid=grid,
        in_specs=[
            pl.BlockSpec((1, 1, N), lambda b, s: (b, 0, 0)),
            pl.BlockSpec((1, 1, N), lambda b, s: (b, 0, 0)),
            pl.BlockSpec((1, 1, N), lambda b, s: (b, 0, 0)),
            pl.BlockSpec((1, SBLK, 1), lambda b, s: (b * nsb + s, 0, 0)),
            pl.BlockSpec((1, SBLK, 1), lambda b, s: (b * nsb + s, 0, 0)),
            pl.BlockSpec((1, SBLK, 1), lambda b, s: (b * nsb + s, 0, 0)),
        ],
        out_specs=[
            pl.BlockSpec((1, SBLK, N), lambda b, s: (b, s, 0)),
            pl.BlockSpec((1, SBLK, 1), lambda b, s: (b * nsb + s, 0, 0)),
        ],
        out_shape=[
            jax.ShapeDtypeStruct((B, S, N), jnp.int32),
            jax.ShapeDtypeStruct((B * nsb, SBLK, 1), jnp.int32),
        ],
    )(x, y, z, cxg, cyg, czg)


XP = 16


def _sc_group_gather(di_f, thr, pts, xyzp, cen):
    info = plsc.get_sparse_core_info()
    NC, NS = info.num_cores, info.num_subcores
    NW = NC * NS
    rpw = (B * S) // NW
    mesh = plsc.VectorSubcoreMesh(core_axis_name="c", subcore_axis_name="s")

    @functools.partial(
        pl.kernel, mesh=mesh,
        compiler_params=pltpu.CompilerParams(
            needs_layout_passes=False, use_tc_tiling_on_sc=False),
        out_type=[jax.ShapeDtypeStruct((B * S, K, D), jnp.float32),
                  jax.ShapeDtypeStruct((B * S, K, XP), jnp.float32)],
        scratch_types=[
            pltpu.VMEM((N,), jnp.int32),
            pltpu.VMEM((rpw,), jnp.int32),
            pltpu.VMEM((K,), jnp.int32),
            pltpu.VMEM((K,), jnp.int32),
            pltpu.VMEM((K, D), jnp.float32),
            pltpu.VMEM((K, XP), jnp.float32),
            pltpu.VMEM((XP,), jnp.float32),
            pltpu.SemaphoreType.DMA,
        ],
    )
    def k(di_hbm, thr_hbm, pts_hbm, xyzp_hbm, cen_hbm, gp_hbm, gx_hbm,
          dirow, thrv_ref, selg, eqbuf, prow, xrow, cenv, sem):
        wid = lax.axis_index("s") * NC + lax.axis_index("c")
        base = wid * rpw
        pltpu.sync_copy(thr_hbm.at[pl.ds(base, rpw)], thrv_ref)
        iota16 = lax.iota(jnp.int32, 16)

        def row_body(r, carry):
            g = base + r
            b = g // S
            bn = b * N
            pltpu.sync_copy(di_hbm.at[g], dirow)
            thrv = plsc.load_gather(thrv_ref, [jnp.full((16,), r, jnp.int32)])

            def chunk(c, offs):
                offlt, offeq = offs
                v = dirow[pl.ds(c * 16, 16)]
                mlt = v < thrv
                meq = v == thrv
                gidx = c * 16 + iota16 + bn
                rlt = offlt + plsc.cumsum(mlt.astype(jnp.int32)) - 1
                plsc.store_scatter(selg, [rlt], gidx, mask=mlt)
                req = offeq + plsc.cumsum(meq.astype(jnp.int32)) - 1
                meq2 = meq & (req < K)
                plsc.store_scatter(eqbuf, [req], gidx, mask=meq2)
                return (offlt + plsc.all_reduce_population_count(mlt),
                        offeq + plsc.all_reduce_population_count(meq2))

            z16 = jnp.zeros((16,), jnp.int32)
            nlt, _ = lax.fori_loop(0, N // 16, chunk, (z16, z16))
            for t in range(K // 16):
                e = eqbuf[pl.ds(t * 16, 16)]
                i = t * 16 + iota16
                plsc.store_scatter(selg, [nlt + i], e, mask=i < (K - nlt))

            pltpu.async_copy(pts_hbm.at[selg], prow, sem).wait()
            pltpu.sync_copy(prow, gp_hbm.at[g])
            pltpu.async_copy(xyzp_hbm.at[selg], xrow, sem).wait()
            pltpu.sync_copy(cen_hbm.at[g], cenv)
            cv = cenv[...]
            for j in range(K):
                xrow[j] = xrow[j] - cv
            pltpu.sync_copy(xrow, gx_hbm.at[g])
            return carry

        lax.fori_loop(0, rpw, row_body, 0)

    return k(di_f, thr, pts, xyzp, cen)


M = B * S * K
MB = 4096


def _stats_pad(y, oc):
    s = jnp.sum(y, axis=0, keepdims=True)
    q = jnp.sum(y * y, axis=0, keepdims=True)
    if oc < 128:
        z = jnp.zeros((1, 128 - oc), jnp.float32)
        s = jnp.concatenate([s, z], axis=1)
        q = jnp.concatenate([q, z], axis=1)
    return jnp.concatenate([s, q, jnp.zeros((6, 128), jnp.float32)], axis=0)


def _acc_stats(st_ref, st):
    @pl.when(pl.program_id(0) == 0)
    def _():
        st_ref[...] = jnp.zeros((8, 128), jnp.float32)

    st_ref[...] += st


def _l1_body(gp_ref, gx_ref, wp_ref, wx_ref, y_ref, st_ref):
    y = jnp.dot(gp_ref[...], wp_ref[...], preferred_element_type=jnp.float32)
    y = y + jnp.dot(gx_ref[...], wx_ref[...],
                    preferred_element_type=jnp.float32)
    y_ref[...] = y
    _acc_stats(st_ref, _stats_pad(y, 64))


def _layer1(gp, gx, wp, wx):
    return pl.pallas_call(
        _l1_body,
        grid=(M // MB,),
        in_specs=[
            pl.BlockSpec((MB, D), lambda i: (i, 0)),
            pl.BlockSpec((MB, XP), lambda i: (i, 0)),
            pl.BlockSpec((D, 64), lambda i: (0, 0)),
            pl.BlockSpec((XP, 64), lambda i: (0, 0)),
        ],
        out_specs=[
            pl.BlockSpec((MB, 64), lambda i: (i, 0)),
            pl.BlockSpec((8, 128), lambda i: (0, 0)),
        ],
        out_shape=[
            jax.ShapeDtypeStruct((M, 64), jnp.float32),
            jax.ShapeDtypeStruct((8, 128), jnp.float32),
        ],
    )(gp, gx, wp, wx)


def _bn_affine(st, g, be, ic):
    mean = st[0:1, :ic] * (1.0 / M)
    var = st[1:2, :ic] * (1.0 / M) - mean * mean
    a = g / jnp.sqrt(var + EPS)
    c = be - mean * a
    return a, c


def _mid_body(ic, oc, y_ref, st_ref, g_ref, be_ref, w_ref, o_ref, so_ref):
    a, c = _bn_affine(st_ref[...], g_ref[...], be_ref[...], ic)
    z = jnp.maximum(y_ref[...] * a + c, 0.0)
    o = jnp.dot(z, w_ref[...], preferred_element_type=jnp.float32)
    o_ref[...] = o
    _acc_stats(so_ref, _stats_pad(o, oc))


def _mid_layer(y, st, g, be, w, ic, oc):
    return pl.pallas_call(
        functools.partial(_mid_body, ic, oc),
        grid=(M // MB,),
        in_specs=[
            pl.BlockSpec((MB, ic), lambda i: (i, 0)),
            pl.BlockSpec((8, 128), lambda i: (0, 0)),
            pl.BlockSpec((1, ic), lambda i: (0, 0)),
            pl.BlockSpec((1, ic), lambda i: (0, 0)),
            pl.BlockSpec((ic, oc), lambda i: (0, 0)),
        ],
        out_specs=[
            pl.BlockSpec((MB, oc), lambda i: (i, 0)),
            pl.BlockSpec((8, 128), lambda i: (0, 0)),
        ],
        out_shape=[
            jax.ShapeDtypeStruct((M, oc), jnp.float32),
            jax.ShapeDtypeStruct((8, 128), jnp.float32),
        ],
    )(y, st, g, be, w)


def _pool_body(y_ref, st_ref, g_ref, be_ref, o_ref):
    a, c = _bn_affine(st_ref[...], g_ref[...], be_ref[...], 128)
    z = jnp.maximum(y_ref[...] * a + c, 0.0)
    o_ref[...] = jnp.max(z.reshape(MB // K, K, 128), axis=1)


def _pool_layer(y, st, g, be):
    return pl.pallas_call(
        _pool_body,
        grid=(M // MB,),
        in_specs=[
            pl.BlockSpec((MB, 128), lambda i: (i, 0)),
            pl.BlockSpec((8, 128), lambda i: (0, 0)),
            pl.BlockSpec((1, 128), lambda i: (0, 0)),
            pl.BlockSpec((1, 128), lambda i: (0, 0)),
        ],
        out_specs=pl.BlockSpec((MB // K, 128), lambda i: (i, 0)),
        out_shape=jax.ShapeDtypeStruct((M // K, 128), jnp.float32),
    )(y, st, g, be)


def kernel(xyz, points, params):
    x = xyz[:, :, 0]
    y = xyz[:, :, 1]
    z = xyz[:, :, 2]
    nx, ny, nz = _fps(x, y, z)
    new_xyz = jnp.stack([nx, ny, nz], axis=-1)

    def _cg(a):
        return a.reshape(B * (S // SBLK), SBLK, 1)

    di, thrg = _knn_thresh(x[:, None, :], y[:, None, :], z[:, None, :],
                           _cg(nx), _cg(ny), _cg(nz))
    thr = thrg.reshape(B, S)

    xyzp = jnp.pad(xyz, ((0, 0), (0, 0), (0, XP - 3))).reshape(B * N, XP)
    cen = jnp.pad(new_xyz, ((0, 0), (0, 0), (0, XP - 3))).reshape(B * S, XP)
    gp, gx = _sc_group_gather(
        di.reshape(B * S, N), thr.reshape(B * S), points.reshape(B * N, D),
        xyzp, cen)
    w0 = params["w0"]
    wx = jnp.pad(w0[:, :3].T, ((0, XP - 3), (0, 0)))
    wp = w0[:, 3:].T
    y1, st1 = _layer1(gp.reshape(M, D), gx.reshape(M, XP), wp, wx)
    y2, st2 = _mid_layer(y1, st1, params["g0"][None, :], params["be0"][None, :],
                         params["w1"].T, 64, 64)
    y3, st3 = _mid_layer(y2, st2, params["g1"][None, :], params["be1"][None, :],
                         params["w2"].T, 64, 128)
    out = _pool_layer(y3, st3, params["g2"][None, :], params["be2"][None, :])
    return (new_xyz, out.reshape(B, S, 128))


# speedup vs baseline: 1.3476x; 1.0458x over previous
import functools

import jax
import jax.numpy as jnp
from jax import lax
from jax.experimental import pallas as pl
from jax.experimental.pallas import tpu as pltpu
from jax.experimental.pallas import tpu_sc as plsc

B = 8
N = 4096
S = 512
K = 32
D = 64
MLP_CH = [64, 64, 128]
EPS = 1e-5


def _fps_body(x_ref, y_ref, z_ref, nx_ref, ny_ref, nz_ref, dist_ref):
    x = x_ref[...]
    y = y_ref[...]
    z = z_ref[...]
    iota = jax.lax.broadcasted_iota(jnp.int32, (B, N), 1)
    lane = jax.lax.broadcasted_iota(jnp.int32, (B, 128), 1)
    dist_ref[...] = jnp.full((B, N), 1e10, jnp.float32)

    def body(i, state):
        far, bx, by, bz = state
        onehot = iota == far
        cx = jnp.max(jnp.where(onehot, x, -jnp.inf), axis=1, keepdims=True)
        cy = jnp.max(jnp.where(onehot, y, -jnp.inf), axis=1, keepdims=True)
        cz = jnp.max(jnp.where(onehot, z, -jnp.inf), axis=1, keepdims=True)
        sel = lane == i
        bx = jnp.where(sel, cx, bx)
        by = jnp.where(sel, cy, by)
        bz = jnp.where(sel, cz, bz)
        dx = x - cx
        dy = y - cy
        dz = z - cz
        d = dx * dx + dy * dy + dz * dz
        dmin = jnp.minimum(dist_ref[...], d)
        dist_ref[...] = dmin
        m = jnp.max(dmin, axis=1, keepdims=True)
        far_new = jnp.min(jnp.where(dmin == m, iota, N), axis=1, keepdims=True)
        return far_new, bx, by, bz

    far = jnp.zeros((B, 1), jnp.int32)
    zbuf = jnp.zeros((B, 128), jnp.float32)
    for j in range(S // 128):
        far, bx, by, bz = jax.lax.fori_loop(0, 128, body, (far, zbuf, zbuf, zbuf))
        nx_ref[:, j * 128:(j + 1) * 128] = bx
        ny_ref[:, j * 128:(j + 1) * 128] = by
        nz_ref[:, j * 128:(j + 1) * 128] = bz


def _fps(x, y, z):
    out = pl.pallas_call(
        _fps_body,
        out_shape=[jax.ShapeDtypeStruct((B, S), jnp.float32)] * 3,
        scratch_shapes=[pltpu.VMEM((B, N), jnp.float32)],
    )(x, y, z)
    return out


SBLK = 128


NCH = N // 16


def _knn_body(x_ref, y_ref, z_ref, xp_ref, yp_ref, zp_ref,
              cx_ref, cy_ref, cz_ref, di_ref, cm_ref, thr_ref):
    cx = cx_ref[0]
    cy = cy_ref[0]
    cz = cz_ref[0]
    dx = x_ref[0] - cx
    dy = y_ref[0] - cy
    dz = z_ref[0] - cz
    d = dx * dx + dy * dy + dz * dz
    di_ref[...] = jax.lax.bitcast_convert_type(d, jnp.int32)[None]
    dpx = xp_ref[0] - cx
    dpy = yp_ref[0] - cy
    dpz = zp_ref[0] - cz
    dp = dpx * dpx + dpy * dpy + dpz * dpz
    cmv = dp[:, 0:NCH]
    for j in range(1, 16):
        cmv = jnp.minimum(cmv, dp[:, j * NCH:(j + 1) * NCH])
    cm_ref[...] = jax.lax.bitcast_convert_type(cmv, jnp.int32)[None]
    dip = jax.lax.bitcast_convert_type(dp, jnp.int32)
    acc = jnp.zeros((SBLK, 1), jnp.int32)
    for b in range(30, -1, -1):
        trial = acc | (1 << b)
        cnt = jnp.sum((dip < trial).astype(jnp.int32), axis=1, keepdims=True)
        acc = jnp.where(cnt < K, trial, acc)
    thr_ref[...] = acc[None]


def _knn_thresh(x, y, z, xp, yp, zp, cxg, cyg, czg):
    nsb = S // SBLK
    grid = (B, nsb)
    rowspec = pl.BlockSpec((1, 1, N), lambda b, s: (b, 0, 0))
    cspec = pl.BlockSpec((1, SBLK, 1), lambda b, s: (b * nsb + s, 0, 0))
    return pl.pallas_call(
        _knn_body,
        grid=grid,
        in_specs=[rowspec] * 6 + [cspec] * 3,
        out_specs=[
            pl.BlockSpec((1, SBLK, N), lambda b, s: (b, s, 0)),
            pl.BlockSpec((1, SBLK, NCH), lambda b, s: (b * nsb + s, 0, 0)),
            pl.BlockSpec((1, SBLK, 1), lambda b, s: (b * nsb + s, 0, 0)),
        ],
        out_shape=[
            jax.ShapeDtypeStruct((B, S, N), jnp.int32),
            jax.ShapeDtypeStruct((B * nsb, SBLK, NCH), jnp.int32),
            jax.ShapeDtypeStruct((B * nsb, SBLK, 1), jnp.int32),
        ],
    )(x, y, z, xp, yp, zp, cxg, cyg, czg)


XP = 16


NFD = 64


def _sc_group_gather(dic, cm, thr, pts, xyzp, cen):
    info = plsc.get_sparse_core_info()
    NC, NS = info.num_cores, info.num_subcores
    NW = NC * NS
    rpw = (B * S) // NW
    mesh = plsc.VectorSubcoreMesh(core_axis_name="c", subcore_axis_name="s")

    @functools.partial(
        pl.kernel, mesh=mesh,
        compiler_params=pltpu.CompilerParams(
            needs_layout_passes=False, use_tc_tiling_on_sc=False),
        out_type=[jax.ShapeDtypeStruct((B * S, K, D), jnp.float32),
                  jax.ShapeDtypeStruct((B * S, K, XP), jnp.float32)],
        scratch_types=[
            pltpu.VMEM((NCH,), jnp.int32),
            pltpu.VMEM((NCH,), jnp.int32),
            pltpu.VMEM((NCH, 16), jnp.int32),
            pltpu.VMEM((rpw,), jnp.int32),
            pltpu.VMEM((K,), jnp.int32),
            pltpu.VMEM((K,), jnp.int32),
            pltpu.VMEM((K, D), jnp.float32),
            pltpu.VMEM((K, XP), jnp.float32),
            pltpu.VMEM((XP,), jnp.float32),
            pltpu.SemaphoreType.DMA,
        ],
    )
    def k(dic_hbm, cm_hbm, thr_hbm, pts_hbm, xyzp_hbm, cen_hbm, gp_hbm,
          gx_hbm, cmrow, cidxg, dch, thrv_ref, selg, eqbuf, prow, xrow,
          cenv, sem):
        wid = lax.axis_index("s") * NC + lax.axis_index("c")
        base = wid * rpw
        pltpu.sync_copy(thr_hbm.at[pl.ds(base, rpw)], thrv_ref)
        iota16 = lax.iota(jnp.int32, 16)
        z16 = jnp.zeros((16,), jnp.int32)
        for t in range(NCH // 16):
            cidxg[pl.ds(t * 16, 16)] = z16

        def row_body(r, carry):
            g = base + r
            b = g // S
            bn = b * N
            pltpu.sync_copy(cm_hbm.at[g], cmrow)
            thrv = plsc.load_gather(thrv_ref, [jnp.full((16,), r, jnp.int32)])

            def scan_cm(t, offc):
                v = cmrow[pl.ds(t * 16, 16)]
                m = v <= thrv
                rank = offc + plsc.cumsum(m.astype(jnp.int32)) - 1
                plsc.store_scatter(cidxg, [rank],
                                   g * NCH + t * 16 + iota16, mask=m)
                return offc + plsc.all_reduce_population_count(m)

            offc = lax.fori_loop(0, NCH // 16, scan_cm, z16)
            nf = jnp.max(offc)

            def dma_small(_):
                pltpu.async_copy(dic_hbm.at[cidxg.at[pl.ds(0, NFD)]],
                                 dch.at[pl.ds(0, NFD)], sem).wait()
                return 0

            def dma_big(_):
                pltpu.async_copy(dic_hbm.at[cidxg], dch, sem).wait()
                return 0

            lax.cond(nf <= NFD, dma_small, dma_big, 0)

            def chunk(j, offs):
                offlt, offeq = offs
                jj = jnp.full((16,), j, jnp.int32)
                v = plsc.load_gather(dch, [jj, iota16])
                cid = plsc.load_gather(cidxg, [jj]) - g * NCH
                mlt = v < thrv
                meq = v == thrv
                gidx = cid * 16 + iota16 + bn
                rlt = offlt + plsc.cumsum(mlt.astype(jnp.int32)) - 1
                plsc.store_scatter(selg, [rlt], gidx, mask=mlt)
                req = offeq + plsc.cumsum(meq.astype(jnp.int32)) - 1
                meq2 = meq & (req < K)
                plsc.store_scatter(eqbuf, [req], gidx, mask=meq2)
                return (offlt + plsc.all_reduce_population_count(mlt),
                        offeq + plsc.all_reduce_population_count(meq2))

            nlt, _ = lax.fori_loop(0, nf, chunk, (z16, z16))
            for t in range(K // 16):
                e = eqbuf[pl.ds(t * 16, 16)]
                i = t * 16 + iota16
                plsc.store_scatter(selg, [nlt + i], e, mask=i < (K - nlt))

            pltpu.async_copy(pts_hbm.at[selg], prow, sem).wait()
            pltpu.sync_copy(prow, gp_hbm.at[g])
            pltpu.async_copy(xyzp_hbm.at[selg], xrow, sem).wait()
            pltpu.sync_copy(cen_hbm.at[g], cenv)
            cv = cenv[...]
            for j in range(K):
                xrow[j] = xrow[j] - cv
            pltpu.sync_copy(xrow, gx_hbm.at[g])
            return carry

        lax.fori_loop(0, rpw, row_body, 0)

    return k(dic, cm, thr, pts, xyzp, cen)


M = B * S * K
MB = 4096


def _stats_pad(y, oc):
    s = jnp.sum(y, axis=0, keepdims=True)
    q = jnp.sum(y * y, axis=0, keepdims=True)
    if oc < 128:
        z = jnp.zeros((1, 128 - oc), jnp.float32)
        s = jnp.concatenate([s, z], axis=1)
        q = jnp.concatenate([q, z], axis=1)
    return jnp.concatenate([s, q, jnp.zeros((6, 128), jnp.float32)], axis=0)


def _acc_stats(st_ref, st):
    @pl.when(pl.program_id(0) == 0)
    def _():
        st_ref[...] = jnp.zeros((8, 128), jnp.float32)

    st_ref[...] += st


def _l1_body(gp_ref, gx_ref, wp_ref, wx_ref, y_ref, st_ref):
    y = jnp.dot(gp_ref[...], wp_ref[...], preferred_element_type=jnp.float32)
    y = y + jnp.dot(gx_ref[...], wx_ref[...],
                    preferred_element_type=jnp.float32)
    y_ref[...] = y
    _acc_stats(st_ref, _stats_pad(y, 64))


def _layer1(gp, gx, wp, wx):
    return pl.pallas_call(
        _l1_body,
        grid=(M // MB,),
        in_specs=[
            pl.BlockSpec((MB, D), lambda i: (i, 0)),
            pl.BlockSpec((MB, XP), lambda i: (i, 0)),
            pl.BlockSpec((D, 64), lambda i: (0, 0)),
            pl.BlockSpec((XP, 64), lambda i: (0, 0)),
        ],
        out_specs=[
            pl.BlockSpec((MB, 64), lambda i: (i, 0)),
            pl.BlockSpec((8, 128), lambda i: (0, 0)),
        ],
        out_shape=[
            jax.ShapeDtypeStruct((M, 64), jnp.float32),
            jax.ShapeDtypeStruct((8, 128), jnp.float32),
        ],
    )(gp, gx, wp, wx)


def _bn_affine(st, g, be, ic):
    mean = st[0:1, :ic] * (1.0 / M)
    var = st[1:2, :ic] * (1.0 / M) - mean * mean
    a = g / jnp.sqrt(var + EPS)
    c = be - mean * a
    return a, c


def _mid_body(ic, oc, y_ref, st_ref, g_ref, be_ref, w_ref, o_ref, so_ref):
    a, c = _bn_affine(st_ref[...], g_ref[...], be_ref[...], ic)
    z = jnp.maximum(y_ref[...] * a + c, 0.0)
    o = jnp.dot(z, w_ref[...], preferred_element_type=jnp.float32)
    o_ref[...] = o
    _acc_stats(so_ref, _stats_pad(o, oc))


def _mid_layer(y, st, g, be, w, ic, oc):
    return pl.pallas_call(
        functools.partial(_mid_body, ic, oc),
        grid=(M // MB,),
        in_specs=[
            pl.BlockSpec((MB, ic), lambda i: (i, 0)),
            pl.BlockSpec((8, 128), lambda i: (0, 0)),
            pl.BlockSpec((1, ic), lambda i: (0, 0)),
            pl.BlockSpec((1, ic), lambda i: (0, 0)),
            pl.BlockSpec((ic, oc), lambda i: (0, 0)),
        ],
        out_specs=[
            pl.BlockSpec((MB, oc), lambda i: (i, 0)),
            pl.BlockSpec((8, 128), lambda i: (0, 0)),
        ],
        out_shape=[
            jax.ShapeDtypeStruct((M, oc), jnp.float32),
            jax.ShapeDtypeStruct((8, 128), jnp.float32),
        ],
    )(y, st, g, be, w)


def _pool_body(y_ref, st_ref, g_ref, be_ref, o_ref):
    a, c = _bn_affine(st_ref[...], g_ref[...], be_ref[...], 128)
    z = jnp.maximum(y_ref[...] * a + c, 0.0)
    o_ref[...] = jnp.max(z.reshape(MB // K, K, 128), axis=1)


def _pool_layer(y, st, g, be):
    return pl.pallas_call(
        _pool_body,
        grid=(M // MB,),
        in_specs=[
            pl.BlockSpec((MB, 128), lambda i: (i, 0)),
            pl.BlockSpec((8, 128), lambda i: (0, 0)),
            pl.BlockSpec((1, 128), lambda i: (0, 0)),
            pl.BlockSpec((1, 128), lambda i: (0, 0)),
        ],
        out_specs=pl.BlockSpec((MB // K, 128), lambda i: (i, 0)),
        out_shape=jax.ShapeDtypeStruct((M // K, 128), jnp.float32),
    )(y, st, g, be)


def kernel(xyz, points, params):
    x = xyz[:, :, 0]
    y = xyz[:, :, 1]
    z = xyz[:, :, 2]
    nx, ny, nz = _fps(x, y, z)
    new_xyz = jnp.stack([nx, ny, nz], axis=-1)

    def _cg(a):
        return a.reshape(B * (S // SBLK), SBLK, 1)

    def _pm(a):
        return a.reshape(B, NCH, 16).swapaxes(1, 2).reshape(B, 1, N)

    di, cm, thrg = _knn_thresh(
        x[:, None, :], y[:, None, :], z[:, None, :],
        _pm(x), _pm(y), _pm(z), _cg(nx), _cg(ny), _cg(nz))

    xyzp = jnp.pad(xyz, ((0, 0), (0, 0), (0, XP - 3))).reshape(B * N, XP)
    cen = jnp.pad(new_xyz, ((0, 0), (0, 0), (0, XP - 3))).reshape(B * S, XP)
    gp, gx = _sc_group_gather(
        di.reshape(B * S * NCH, 16), cm.reshape(B * S, NCH),
        thrg.reshape(B * S), points.reshape(B * N, D), xyzp, cen)
    w0 = params["w0"]
    wx = jnp.pad(w0[:, :3].T, ((0, XP - 3), (0, 0)))
    wp = w0[:, 3:].T
    y1, st1 = _layer1(gp.reshape(M, D), gx.reshape(M, XP), wp, wx)
    y2, st2 = _mid_layer(y1, st1, params["g0"][None, :], params["be0"][None, :],
                         params["w1"].T, 64, 64)
    y3, st3 = _mid_layer(y2, st2, params["g1"][None, :], params["be1"][None, :],
                         params["w2"].T, 64, 128)
    out = _pool_layer(y3, st3, params["g2"][None, :], params["be2"][None, :])
    return (new_xyz, out.reshape(B, S, 128))
